# baseline (speedup 1.0000x reference)
import functools

import jax
import jax.numpy as jnp
from jax import lax
from jax.experimental import pallas as pl
from jax.experimental.pallas import tpu as pltpu
from jax.experimental.pallas import tpu_sc as plsc

N = 10000
E = 320000
D = 128
HEAD = 4
HD = 32
HH = 128
NP = 10240
BLK = 512
NBLK = NP // BLK
WACC = 136
NC = 2
NS = 16
NW = NC * NS
EPW = 10240
EPAD = EPW * NW
EB = 64
NEB = EPW // EB
TROWS = NP // NS


def _prec_body(h_ref, wd_ref, bd_ref, ww_ref, bw_ref, wt1_ref, wt2_ref,
               wd1_ref, bad_ref, hw_ref, ad_ref):
    hb = h_ref[...]
    hl = jnp.dot(hb, wd_ref[...], preferred_element_type=jnp.float32) + bd_ref[...]
    t = jnp.tanh(2.0 * hl)
    hw = jnp.dot(hb, ww_ref[0], preferred_element_type=jnp.float32) + bw_ref[0]
    tl = (jnp.dot(hw, wt1_ref[0], preferred_element_type=jnp.float32)
          + jnp.dot(t, wt2_ref[0], preferred_element_type=jnp.float32))
    ad = jnp.dot(hw, wd1_ref[0], preferred_element_type=jnp.float32) + bad_ref[0]
    hw_ref[0] = jnp.concatenate([hw, tl], axis=1)
    ad_ref[0] = ad


def _precompute(h_p, wd, bd2, ww3, bw3, wt1, wt2, wd1, bad3):
    f32 = jnp.float32
    return pl.pallas_call(
        _prec_body,
        grid=(3, NBLK),
        in_specs=[
            pl.BlockSpec((BLK, D), lambda r, i: (i, 0)),
            pl.BlockSpec((D, D), lambda r, i: (0, 0)),
            pl.BlockSpec((1, D), lambda r, i: (0, 0)),
            pl.BlockSpec((1, D, D), lambda r, i: (r, 0, 0)),
            pl.BlockSpec((1, 1, D), lambda r, i: (r, 0, 0)),
            pl.BlockSpec((1, D, 16), lambda r, i: (r, 0, 0)),
            pl.BlockSpec((1, D, 16), lambda r, i: (r, 0, 0)),
            pl.BlockSpec((1, D, 16), lambda r, i: (r, 0, 0)),
            pl.BlockSpec((1, 1, 16), lambda r, i: (r, 0, 0)),
        ],
        out_specs=[
            pl.BlockSpec((1, BLK, D + 16), lambda r, i: (r, i, 0)),
            pl.BlockSpec((1, BLK, 16), lambda r, i: (r, i, 0)),
        ],
        out_shape=[
            jax.ShapeDtypeStruct((3, NP, D + 16), f32),
            jax.ShapeDtypeStruct((3, NP, 16), f32),
        ],
    )(h_p, wd, bd2, ww3, bw3, wt1, wt2, wd1, bad3)


def _sc_body(hw0, hw1, hw2, ad0, ad1, ad2,
             s0, s1, s2, d0, d1, d2, zac, acc_out,
             sidx0, sidx1, didx0, didx1, didx2, didx3,
             hwrows0, hwrows1, adrows0, adrows1,
             contrib0, contrib1, acc,
             sis0, sis1, sid0, sid1, sid2, sid3,
             smh0, smh1, sma0, sma1, ssc0, ssc1):
    cid = lax.axis_index("c")
    sid = lax.axis_index("s")
    wid = sid * NC + cid
    hws = (hw0, hw1, hw2)
    ads = (ad0, ad1, ad2)
    srcs = (s0, s1, s2)
    dsts = (d0, d1, d2)
    sidx = (sidx0, sidx1)
    didx = (didx0, didx1, didx2, didx3)
    hwrows = (hwrows0, hwrows1)
    adrows = (adrows0, adrows1)
    contrib = (contrib0, contrib1)
    sis = (sis0, sis1)
    sdi = (sid0, sid1, sid2, sid3)
    smh = (smh0, smh1)
    sma = (sma0, sma1)
    ssc = (ssc0, ssc1)
    lane = lax.iota(jnp.int32, 16)
    row0 = sid * TROWS
    for rel in range(3):
        src_r, dst_r = srcs[rel], dsts[rel]
        hw_r, ad_r = hws[rel], ads[rel]
        base_w = wid * EPW

        pltpu.sync_copy(zac.at[pl.ds(row0, TROWS)], acc.at[pl.ds(row0, TROWS)])
        plsc.subcore_barrier()

        def issue_idx(b, p2, p4):
            base = base_w + b * EB
            pltpu.async_copy(src_r.at[pl.ds(base, EB)], sidx[p2], sis[p2])
            pltpu.async_copy(dst_r.at[pl.ds(base, EB)], didx[p4], sdi[p4])

        def issue_gathers(p2, p4):
            pltpu.make_async_copy(src_r.at[pl.ds(0, EB)], sidx[p2], sis[p2]).wait()
            pltpu.make_async_copy(dst_r.at[pl.ds(0, EB)], didx[p4], sdi[p4]).wait()
            pltpu.async_copy(hw_r.at[sidx[p2]], hwrows[p2], smh[p2])
            pltpu.async_copy(ad_r.at[didx[p4]], adrows[p2], sma[p2])

        def wait_scatter(p2, p4):
            pltpu.make_async_copy(
                contrib[p2], acc.at[didx[p4]], ssc[p2]).wait()

        def compute_block(p2):
            hwb, adb, ctb = hwrows[p2], adrows[p2], contrib[p2]
            pltpu.make_async_copy(hw_r.at[sidx[p2]], hwb, smh[p2]).wait()
            pltpu.make_async_copy(ad_r.at[sidx[p2]], adb, sma[p2]).wait()

            @plsc.parallel_loop(0, EB, 1, unroll=8)
            def edge_body(r):
                tl16 = hwb[r, pl.ds(D, 16)]
                ad16 = adb[r, :]
                ssum = tl16 + ad16
                a = jnp.where(ssum >= 0.0, ssum, 0.01 * ssum)
                e = jnp.exp(a)
                comb = jnp.where(lane < 4, e,
                                 jnp.where(lane == 4, tl16,
                                           jnp.where(lane == 5, 1.0, 0.0)))
                ctb[r, pl.ds(0, 16)] = comb
                for h in range(HEAD):
                    hidx = jnp.full((16,), h, jnp.int32)
                    eb = lax.gather(
                        e, hidx[:, None],
                        lax.GatherDimensionNumbers(
                            offset_dims=(), collapsed_slice_dims=(0,),
                            start_index_map=(0,)),
                        slice_sizes=(1,),
                        mode=lax.GatherScatterMode.PROMISE_IN_BOUNDS)
                    for q in range(2):
                        c0 = h * HD + q * 16
                        ctb[r, pl.ds(8 + c0, 16)] = hwb[r, pl.ds(c0, 16)] * eb

        issue_idx(0, 0, 0)
        issue_idx(1, 1, 1)
        issue_gathers(0, 0)

        def quad_body(g, _):
            b0 = g * 4
            for p in range(4):
                b = b0 + p
                p2 = p & 1
                p4 = p

                @pl.when(b >= 2)
                def _():
                    wait_scatter(p2, (p + 2) & 3)
                @pl.when(b + 1 < NEB)
                def _():
                    issue_gathers((p + 1) & 1, (p + 1) & 3)
                compute_block(p2)
                pltpu.async_copy(contrib[p2], acc.at[didx[p4]], ssc[p2],
                                 add=True)

                @pl.when(b + 2 < NEB)
                def _():
                    issue_idx(b + 2, p2, (p + 2) & 3)
            return 0

        lax.fori_loop(0, NEB // 4, quad_body, 0)
        wait_scatter(0, 2)
        wait_scatter(1, 3)
        plsc.subcore_barrier()
        pltpu.sync_copy(acc.at[pl.ds(row0, TROWS)],
                        acc_out.at[rel, cid, pl.ds(row0, TROWS)])


def _sc_edges(hw3, ad3, srcs, dsts, zac):
    f32 = jnp.float32
    mesh = plsc.VectorSubcoreMesh(core_axis_name="c", subcore_axis_name="s")
    fn = pl.kernel(
        _sc_body,
        out_type=jax.ShapeDtypeStruct((3, NC, NP, WACC), f32),
        mesh=mesh,
        compiler_params=pltpu.CompilerParams(use_tc_tiling_on_sc=False),
        scratch_types=(
            [pltpu.VMEM((EB,), jnp.int32)] * 2
            + [pltpu.VMEM((EB,), jnp.int32)] * 4
            + [pltpu.VMEM((EB, D + 16), f32)] * 2
            + [pltpu.VMEM((EB, 16), f32)] * 2
            + [pltpu.VMEM((EB, WACC), f32)] * 2
            + [pltpu.VMEM_SHARED((NP, WACC), f32)]
            + [pltpu.SemaphoreType.DMA] * 12
        ),
    )
    return fn(hw3[0], hw3[1], hw3[2], ad3[0], ad3[1], ad3[2],
              srcs[0], srcs[1], srcs[2], dsts[0], dsts[1], dsts[2], zac)


def _fin_body(acc_ref, hw_ref, b1_ref, wl_ref, bl_ref, y_ref):
    r = pl.program_id(1)
    A = acc_ref[0, 0] + acc_ref[0, 1]
    num = A[:, 8:136]
    den4 = A[:, 0:4]
    gs = A[:, 4:5]
    cnt = A[:, 5:6]
    hrow = lax.broadcasted_iota(jnp.int32, (HEAD, HH), 0)
    ccol = lax.broadcasted_iota(jnp.int32, (HEAD, HH), 1)
    s4t = (ccol // HD == hrow).astype(jnp.float32)
    den = jnp.dot(den4, s4t, preferred_element_type=jnp.float32)
    out = num / jnp.where(den == 0.0, 1.0, den)
    es = gs / jnp.maximum(cnt, 1.0)
    dotb = jnp.sum(out * b1_ref[0], axis=1, keepdims=True)
    gate = jax.nn.sigmoid(es + dotb)
    x = gate * out + (1.0 - gate) * hw_ref[0][:, :D]
    contrib = jnp.dot(x, wl_ref[0], preferred_element_type=jnp.float32)

    @pl.when(r == 0)
    def _():
        y_ref[...] = contrib + bl_ref[...]

    @pl.when(r > 0)
    def _():
        y_ref[...] = y_ref[...] + contrib


def _finalize(acc, hw3, b1_3, wl3, bl2):
    return pl.pallas_call(
        _fin_body,
        grid=(NBLK, 3),
        in_specs=[
            pl.BlockSpec((1, NC, BLK, WACC), lambda i, r: (r, 0, i, 0)),
            pl.BlockSpec((1, BLK, D + 16), lambda i, r: (r, i, 0)),
            pl.BlockSpec((1, 1, D), lambda i, r: (r, 0, 0)),
            pl.BlockSpec((1, D, D), lambda i, r: (r, 0, 0)),
            pl.BlockSpec((1, D), lambda i, r: (0, 0)),
        ],
        out_specs=pl.BlockSpec((BLK, D), lambda i, r: (i, 0)),
        out_shape=jax.ShapeDtypeStruct((NP, D), jnp.float32),
    )(acc, hw3, b1_3, wl3, bl2)


def kernel(h, params, edge_index_0, edge_index_1, edge_index_2):
    f32 = jnp.float32
    h_p = jnp.pad(h, ((0, NP - N), (0, 0)))
    wd = params['Wd']
    bd2 = params['bd'].reshape(1, D)
    ww3 = jnp.stack([params['Ww%d' % i] for i in range(3)])
    bw3 = jnp.stack([params['bw%d' % i] for i in range(3)]).reshape(3, 1, D)
    cidx = jnp.arange(HH)
    s4p = (cidx[:, None] // HD == jnp.arange(16)[None, :]).astype(f32)
    col4 = (jnp.arange(16) == 4).astype(f32)
    wt1_l, wt2_l, wd1_l, bad_l, b1_l = [], [], [], [], []
    for i in range(3):
        wa = params['Wa%d' % i][:, 0]
        ba = params['ba%d' % i][0]
        beta = params['beta%d' % i][:, 0]
        wa_s = jnp.tile(wa[0:HD], HEAD)
        wa_d = jnp.tile(wa[HD:2 * HD], HEAD)
        wa_e = jnp.tile(wa[2 * HD:3 * HD], HEAD)
        wt1_l.append(wa_s[:, None] * s4p)
        wt2_l.append(wa_e[:, None] * s4p + beta[0:HH, None] * col4[None, :])
        wd1_l.append(wa_d[:, None] * s4p)
        bad_l.append(ba * (jnp.arange(16) < 4).astype(f32))
        b1_l.append(beta[HH:])
    wt1 = jnp.stack(wt1_l)
    wt2 = jnp.stack(wt2_l)
    wd1 = jnp.stack(wd1_l)
    bad3 = jnp.stack(bad_l).reshape(3, 1, 16)
    b1_3 = jnp.stack(b1_l).reshape(3, 1, D)
    wl3 = params['Wl'].reshape(3, HH, HH)
    bl2 = params['bl'].reshape(1, HH)

    hw3, ad3 = _precompute(h_p, wd, bd2, ww3, bw3, wt1, wt2, wd1, bad3)

    pad = jnp.full((EPAD - E,), NP - 1, jnp.int32)
    srcs = tuple(jnp.concatenate([ei[0], pad])
                 for ei in (edge_index_0, edge_index_1, edge_index_2))
    dsts = tuple(jnp.concatenate([ei[1], pad])
                 for ei in (edge_index_0, edge_index_1, edge_index_2))
    zac = jnp.zeros((NP, WACC), f32)
    acc = _sc_edges(hw3, ad3, srcs, dsts, zac)

    y = _finalize(acc, hw3, b1_3, wl3, bl2)
    return y[:N]


# speedup vs baseline: 1.5310x; 1.5310x over previous
import functools

import jax
import jax.numpy as jnp
from jax import lax
from jax.experimental import pallas as pl
from jax.experimental.pallas import tpu as pltpu
from jax.experimental.pallas import tpu_sc as plsc

N = 10000
E = 320000
D = 128
HEAD = 4
HD = 32
HH = 128
NP = 10240
BLK = 512
NBLK = NP // BLK
WACC = 136
NC = 2
NS = 16
NW = NC * NS
EPW = 10240
EPAD = EPW * NW
EB = 80
NEB = EPW // EB
TROWS = NP // NS

_PI = []
for _g in range(HEAD):
    _PI += [HD * _g + 2 * _j for _j in range(16)]
    _PI += [HD * _g + 2 * _j + 1 for _j in range(16)]
_INV = [0] * HH
for _j, _p in enumerate(_PI):
    _INV[_p] = _j
_COLPERM = tuple(_PI.index(c) for c in range(HH))


def _prec_body(h_ref, wd_ref, bd_ref, ww_ref, bw_ref, wwp_ref, bwp_ref,
               wt1_ref, wt2_ref, wd1_ref, bad_ref,
               hwf_ref, hwb_ref, tl_ref, ad_ref):
    hb = h_ref[...]
    hl = jnp.dot(hb, wd_ref[...], preferred_element_type=jnp.float32) + bd_ref[...]
    t = jnp.tanh(2.0 * hl)
    hw = jnp.dot(hb, ww_ref[0], preferred_element_type=jnp.float32) + bw_ref[0]
    hwp = jnp.dot(hb, wwp_ref[0], preferred_element_type=jnp.float32) + bwp_ref[0]
    tl = (jnp.dot(hw, wt1_ref[0], preferred_element_type=jnp.float32)
          + jnp.dot(t, wt2_ref[0], preferred_element_type=jnp.float32))
    ad = jnp.dot(hw, wd1_ref[0], preferred_element_type=jnp.float32) + bad_ref[0]
    hwf_ref[0] = hw
    hwb_ref[0] = hwp.astype(jnp.bfloat16)
    tl_ref[0] = tl
    ad_ref[0] = ad


def _precompute(h_p, wd, bd2, ww3, bw3, wwp3, bwp3, wt1, wt2, wd1, bad3):
    f32 = jnp.float32
    return pl.pallas_call(
        _prec_body,
        grid=(3, NBLK),
        in_specs=[
            pl.BlockSpec((BLK, D), lambda r, i: (i, 0)),
            pl.BlockSpec((D, D), lambda r, i: (0, 0)),
            pl.BlockSpec((1, D), lambda r, i: (0, 0)),
            pl.BlockSpec((1, D, D), lambda r, i: (r, 0, 0)),
            pl.BlockSpec((1, 1, D), lambda r, i: (r, 0, 0)),
            pl.BlockSpec((1, D, D), lambda r, i: (r, 0, 0)),
            pl.BlockSpec((1, 1, D), lambda r, i: (r, 0, 0)),
            pl.BlockSpec((1, D, 16), lambda r, i: (r, 0, 0)),
            pl.BlockSpec((1, D, 16), lambda r, i: (r, 0, 0)),
            pl.BlockSpec((1, D, 16), lambda r, i: (r, 0, 0)),
            pl.BlockSpec((1, 1, 16), lambda r, i: (r, 0, 0)),
        ],
        out_specs=[
            pl.BlockSpec((1, BLK, D), lambda r, i: (r, i, 0)),
            pl.BlockSpec((1, BLK, D), lambda r, i: (r, i, 0)),
            pl.BlockSpec((1, BLK, 16), lambda r, i: (r, i, 0)),
            pl.BlockSpec((1, BLK, 16), lambda r, i: (r, i, 0)),
        ],
        out_shape=[
            jax.ShapeDtypeStruct((3, NP, D), f32),
            jax.ShapeDtypeStruct((3, NP, D), jnp.bfloat16),
            jax.ShapeDtypeStruct((3, NP, 16), f32),
            jax.ShapeDtypeStruct((3, NP, 16), f32),
        ],
    )(h_p, wd, bd2, ww3, bw3, wwp3, bwp3, wt1, wt2, wd1, bad3)


def _sc_body(hw0, hw1, hw2, tl0, tl1, tl2, ad0, ad1, ad2,
             s0, s1, s2, d0, d1, d2, zac, acc_out,
             sidx0, sidx1, didx0, didx1, didx2, didx3,
             hwrows0, hwrows1, tlrows0, tlrows1, adrows0, adrows1,
             contrib0, contrib1, acc,
             sis0, sis1, sid0, sid1, sid2, sid3,
             smh0, smh1, smt0, smt1, sma0, sma1, ssc0, ssc1):
    cid = lax.axis_index("c")
    sid = lax.axis_index("s")
    wid = sid * NC + cid
    hws = (hw0, hw1, hw2)
    tls = (tl0, tl1, tl2)
    ads = (ad0, ad1, ad2)
    srcs = (s0, s1, s2)
    dsts = (d0, d1, d2)
    sidx = (sidx0, sidx1)
    didx = (didx0, didx1, didx2, didx3)
    hwrows = (hwrows0, hwrows1)
    tlrows = (tlrows0, tlrows1)
    adrows = (adrows0, adrows1)
    contrib = (contrib0, contrib1)
    sis = (sis0, sis1)
    sdi = (sid0, sid1, sid2, sid3)
    smh = (smh0, smh1)
    smt = (smt0, smt1)
    sma = (sma0, sma1)
    ssc = (ssc0, ssc1)
    lane = lax.iota(jnp.int32, 16)
    row0 = sid * TROWS
    for rel in range(3):
        src_r, dst_r = srcs[rel], dsts[rel]
        hw_r, tl_r, ad_r = hws[rel], tls[rel], ads[rel]
        base_w = wid * EPW

        pltpu.sync_copy(zac.at[pl.ds(row0, TROWS)], acc.at[pl.ds(row0, TROWS)])
        plsc.subcore_barrier()

        def issue_idx(b, p2, p4):
            base = base_w + b * EB
            pltpu.async_copy(src_r.at[pl.ds(base, EB)], sidx[p2], sis[p2])
            pltpu.async_copy(dst_r.at[pl.ds(base, EB)], didx[p4], sdi[p4])

        def issue_gathers(p2, p4):
            pltpu.make_async_copy(src_r.at[pl.ds(0, EB)], sidx[p2], sis[p2]).wait()
            pltpu.make_async_copy(dst_r.at[pl.ds(0, EB)], didx[p4], sdi[p4]).wait()
            pltpu.async_copy(hw_r.at[sidx[p2]], hwrows[p2], smh[p2])
            pltpu.async_copy(tl_r.at[sidx[p2]], tlrows[p2], smt[p2])
            pltpu.async_copy(ad_r.at[didx[p4]], adrows[p2], sma[p2])

        def wait_scatter(p2, p4):
            pltpu.make_async_copy(
                contrib[p2], acc.at[didx[p4]], ssc[p2]).wait()

        def compute_block(p2):
            hwb, tlb, adb, ctb = hwrows[p2], tlrows[p2], adrows[p2], contrib[p2]
            pltpu.make_async_copy(hw_r.at[sidx[p2]], hwb, smh[p2]).wait()
            pltpu.make_async_copy(tl_r.at[sidx[p2]], tlb, smt[p2]).wait()
            pltpu.make_async_copy(ad_r.at[sidx[p2]], adb, sma[p2]).wait()

            @plsc.parallel_loop(0, EB, 1, unroll=8)
            def edge_body(r):
                tl16 = tlb[r, :]
                ad16 = adb[r, :]
                ssum = tl16 + ad16
                a = jnp.where(ssum >= 0.0, ssum, 0.01 * ssum)
                e = jnp.exp(a)
                comb = jnp.where(lane < 4, e,
                                 jnp.where(lane == 4, tl16,
                                           jnp.where(lane == 5, 1.0, 0.0)))
                ctb[r, pl.ds(0, 16)] = comb
                for h in range(HEAD):
                    hidx = jnp.full((16,), h, jnp.int32)
                    eb = lax.gather(
                        e, hidx[:, None],
                        lax.GatherDimensionNumbers(
                            offset_dims=(), collapsed_slice_dims=(0,),
                            start_index_map=(0,)),
                        slice_sizes=(1,),
                        mode=lax.GatherScatterMode.PROMISE_IN_BOUNDS)
                    v = hwb[r, pl.ds(h * 16, 16)]
                    va = lax.bitcast_convert_type(
                        lax.shift_left(v, 16), jnp.float32)
                    vb = lax.bitcast_convert_type(
                        lax.bitwise_and(v, jnp.int32(-65536)),
                        jnp.float32)
                    c0 = 8 + h * HD
                    ctb[r, pl.ds(c0, 16)] = va * eb
                    ctb[r, pl.ds(c0 + 16, 16)] = vb * eb

        issue_idx(0, 0, 0)
        issue_idx(1, 1, 1)
        issue_gathers(0, 0)

        def quad_body(g, _):
            b0 = g * 4
            for p in range(4):
                b = b0 + p
                p2 = p & 1
                p4 = p

                @pl.when(b >= 2)
                def _():
                    wait_scatter(p2, (p + 2) & 3)
                @pl.when(b + 1 < NEB)
                def _():
                    issue_gathers((p + 1) & 1, (p + 1) & 3)
                compute_block(p2)
                pltpu.async_copy(contrib[p2], acc.at[didx[p4]], ssc[p2],
                                 add=True)

                @pl.when(b + 2 < NEB)
                def _():
                    issue_idx(b + 2, p2, (p + 2) & 3)
            return 0

        lax.fori_loop(0, NEB // 4, quad_body, 0)
        wait_scatter(0, 2)
        wait_scatter(1, 3)
        plsc.subcore_barrier()
        pltpu.sync_copy(acc.at[pl.ds(row0, TROWS)],
                        acc_out.at[rel, cid, pl.ds(row0, TROWS)])


def _sc_edges(hwb3, tl3, ad3, srcs, dsts, zac):
    f32 = jnp.float32
    mesh = plsc.VectorSubcoreMesh(core_axis_name="c", subcore_axis_name="s")
    fn = pl.kernel(
        _sc_body,
        out_type=jax.ShapeDtypeStruct((3, NC, NP, WACC), f32),
        mesh=mesh,
        compiler_params=pltpu.CompilerParams(use_tc_tiling_on_sc=False),
        scratch_types=(
            [pltpu.VMEM((EB,), jnp.int32)] * 2
            + [pltpu.VMEM((EB,), jnp.int32)] * 4
            + [pltpu.VMEM((EB, D // 2), jnp.int32)] * 2
            + [pltpu.VMEM((EB, 16), f32)] * 2
            + [pltpu.VMEM((EB, 16), f32)] * 2
            + [pltpu.VMEM((EB, WACC), f32)] * 2
            + [pltpu.VMEM_SHARED((NP, WACC), f32)]
            + [pltpu.SemaphoreType.DMA] * 14
        ),
    )
    return fn(hwb3[0], hwb3[1], hwb3[2], tl3[0], tl3[1], tl3[2],
              ad3[0], ad3[1], ad3[2],
              srcs[0], srcs[1], srcs[2], dsts[0], dsts[1], dsts[2], zac)


def _fin_body(acc_ref, hw_ref, b1_ref, wl_ref, bl_ref, y_ref):
    r = pl.program_id(1)
    A = acc_ref[0, 0] + acc_ref[0, 1]
    num = A[:, 8:136]
    den4 = A[:, 0:4]
    gs = A[:, 4:5]
    cnt = A[:, 5:6]
    hrow = lax.broadcasted_iota(jnp.int32, (HEAD, HH), 0)
    ccol = lax.broadcasted_iota(jnp.int32, (HEAD, HH), 1)
    s4t = (ccol // HD == hrow).astype(jnp.float32)
    den = jnp.dot(den4, s4t, preferred_element_type=jnp.float32)
    out = num / jnp.where(den == 0.0, 1.0, den)
    es = gs / jnp.maximum(cnt, 1.0)
    dotb = jnp.sum(out * b1_ref[0], axis=1, keepdims=True)
    gate = jax.nn.sigmoid(es + dotb)
    x = gate * out + (1.0 - gate) * hw_ref[0]
    contrib = jnp.dot(x, wl_ref[0], preferred_element_type=jnp.float32)

    @pl.when(r == 0)
    def _():
        y_ref[...] = contrib + bl_ref[...]

    @pl.when(r > 0)
    def _():
        y_ref[...] = y_ref[...] + contrib


def _finalize(acc, hwf3, b1_3, wl3, bl2):
    return pl.pallas_call(
        _fin_body,
        grid=(NBLK, 3),
        in_specs=[
            pl.BlockSpec((1, NC, BLK, WACC), lambda i, r: (r, 0, i, 0)),
            pl.BlockSpec((1, BLK, D), lambda i, r: (r, i, 0)),
            pl.BlockSpec((1, 1, D), lambda i, r: (r, 0, 0)),
            pl.BlockSpec((1, D, D), lambda i, r: (r, 0, 0)),
            pl.BlockSpec((1, D), lambda i, r: (0, 0)),
        ],
        out_specs=pl.BlockSpec((BLK, D), lambda i, r: (i, 0)),
        out_shape=jax.ShapeDtypeStruct((NP, D), jnp.float32),
    )(acc, hwf3, b1_3, wl3, bl2)


def kernel(h, params, edge_index_0, edge_index_1, edge_index_2):
    f32 = jnp.float32
    h_p = jnp.pad(h, ((0, NP - N), (0, 0)))
    wd = params['Wd']
    bd2 = params['bd'].reshape(1, D)
    ww3 = jnp.stack([params['Ww%d' % i] for i in range(3)])
    bw3 = jnp.stack([params['bw%d' % i] for i in range(3)]).reshape(3, 1, D)
    colperm = jnp.asarray(_COLPERM, jnp.int32)
    wwp3 = ww3[:, :, colperm]
    bwp3 = bw3[:, :, colperm]
    cidx = jnp.arange(HH)
    s4p = (cidx[:, None] // HD == jnp.arange(16)[None, :]).astype(f32)
    col4 = (jnp.arange(16) == 4).astype(f32)
    wt1_l, wt2_l, wd1_l, bad_l, b1_l = [], [], [], [], []
    for i in range(3):
        wa = params['Wa%d' % i][:, 0]
        ba = params['ba%d' % i][0]
        beta = params['beta%d' % i][:, 0]
        wa_s = jnp.tile(wa[0:HD], HEAD)
        wa_d = jnp.tile(wa[HD:2 * HD], HEAD)
        wa_e = jnp.tile(wa[2 * HD:3 * HD], HEAD)
        wt1_l.append(wa_s[:, None] * s4p)
        wt2_l.append(wa_e[:, None] * s4p + beta[0:HH, None] * col4[None, :])
        wd1_l.append(wa_d[:, None] * s4p)
        bad_l.append(ba * (jnp.arange(16) < 4).astype(f32))
        b1_l.append(beta[HH:])
    wt1 = jnp.stack(wt1_l)
    wt2 = jnp.stack(wt2_l)
    wd1 = jnp.stack(wd1_l)
    bad3 = jnp.stack(bad_l).reshape(3, 1, 16)
    b1_3 = jnp.stack(b1_l).reshape(3, 1, D)
    wl3 = params['Wl'].reshape(3, HH, HH)
    bl2 = params['bl'].reshape(1, HH)

    hwf3, hwb3, tl3, ad3 = _precompute(
        h_p, wd, bd2, ww3, bw3, wwp3, bwp3, wt1, wt2, wd1, bad3)
    hwi3 = lax.bitcast_convert_type(
        hwb3.reshape(3, NP, D // 2, 2), jnp.int32)

    pad = jnp.full((EPAD - E,), NP - 1, jnp.int32)
    srcs = tuple(jnp.concatenate([ei[0], pad])
                 for ei in (edge_index_0, edge_index_1, edge_index_2))
    dsts = tuple(jnp.concatenate([ei[1], pad])
                 for ei in (edge_index_0, edge_index_1, edge_index_2))
    zac = jnp.zeros((NP, WACC), f32)
    acc = _sc_edges(hwi3, tl3, ad3, srcs, dsts, zac)

    y = _finalize(acc, hwf3, b1_3, wl3, bl2)
    return y[:N]
